# Initial kernel scaffold; baseline (speedup 1.0000x reference)
import jax
import jax.numpy as jnp
from jax.experimental import pallas as pl


def kernel(x, edge_index, edge_weight, W_pool, b_pool, W_self, b_self, W_neigh, b_neigh):
    raise NotImplementedError("write your pallas kernel here")


# speedup vs baseline: 1.3507x; 1.3507x over previous
import functools

import jax
import jax.numpy as jnp
from jax import lax
from jax.experimental import pallas as pl
from jax.experimental.pallas import tpu as pltpu
from jax.experimental.pallas import tpu_sc as plsc

_N = 10000
_E = 320000
_D = 128

_NC = 2
_NS = 16
_NW = _NC * _NS
_L = 16

_R = 320
_NPAD = _NW * _R
_CHUNK = 4000
_NCHUNK = _E // _CHUNK
_K = 128
_FG = _D // _L

_BLK = 1000


def _mm_relu_body(x_ref, w_ref, b_ref, o_ref):
    acc = jnp.dot(x_ref[...], w_ref[...], preferred_element_type=jnp.float32)
    o_ref[...] = jnp.maximum(acc + b_ref[...], 0.0)


def _final_body(x_ref, n_ref, ws_ref, wn_ref, b_ref, o_ref):
    acc = jnp.dot(x_ref[...], ws_ref[...], preferred_element_type=jnp.float32)
    acc = acc + jnp.dot(n_ref[...], wn_ref[...], preferred_element_type=jnp.float32)
    o_ref[...] = acc + b_ref[...]


def _sc_body(h_hbm, src_hbm, dst_hbm, w_hbm, out_hbm,
             acc, dstc, srcc, wc, gidx, locb, wbuf, rows, sem):
    wid = lax.axis_index("s") * _NC + lax.axis_index("c")
    lo = wid * _R

    def _zero(i, _):
        acc[pl.ds(i * _L, _L)] = jnp.zeros((_L,), jnp.float32)
        return 0
    lax.fori_loop(0, (_R * _D) // _L, _zero, 0)
    for q in range((_K + _L) // _L):
        gidx[pl.ds(q * _L, _L)] = jnp.zeros((_L,), jnp.int32)
        locb[pl.ds(q * _L, _L)] = jnp.zeros((_L,), jnp.int32)

    def _flush(n):
        pltpu.async_copy(h_hbm.at[gidx.at[pl.ds(0, _K)]], rows, sem).wait()

        def _edge(i, _):
            loc = locb[pl.ds(i, _L)][0]
            wv = wbuf[pl.ds(i, _L)][0]
            base = loc * _D
            for f in range(_FG):
                a = acc[pl.ds(base + f * _L, _L)]
                r = rows[i, pl.ds(f * _L, _L)]
                acc[pl.ds(base + f * _L, _L)] = jnp.maximum(a, r * wv)
            return 0
        lax.fori_loop(0, n, _edge, 0)

    def _chunk(c, fill):
        cbase = c * _CHUNK
        pltpu.sync_copy(dst_hbm.at[pl.ds(cbase, _CHUNK)], dstc)
        pltpu.sync_copy(src_hbm.at[pl.ds(cbase, _CHUNK)], srcc)
        pltpu.sync_copy(w_hbm.at[pl.ds(cbase, _CHUNK)], wc)

        def _group(j, fill):
            d16 = dstc[pl.ds(j * _L, _L)]
            s16 = srcc[pl.ds(j * _L, _L)]
            w16 = wc[pl.ds(j * _L, _L)]
            m = (d16 >= lo) & (d16 < lo + _R)
            incl = plsc.cumsum(m.astype(jnp.int32))
            plsc.store_compressed(gidx.at[pl.ds(fill, _L)], s16, mask=m)
            plsc.store_compressed(locb.at[pl.ds(fill, _L)], d16 - lo, mask=m)
            plsc.store_compressed(wbuf.at[pl.ds(fill, _L)], w16, mask=m)
            fill = fill + incl[_L - 1]

            @pl.when(fill >= _K)
            def _():
                _flush(_K)
                gidx[pl.ds(0, _L)] = gidx[pl.ds(_K, _L)]
                locb[pl.ds(0, _L)] = locb[pl.ds(_K, _L)]
                wbuf[pl.ds(0, _L)] = wbuf[pl.ds(_K, _L)]
            return jnp.where(fill >= _K, fill - _K, fill)

        return lax.fori_loop(0, _CHUNK // _L, _group, fill)

    fill = lax.fori_loop(0, _NCHUNK, _chunk, jnp.int32(0))
    _flush(fill)
    pltpu.sync_copy(acc, out_hbm.at[pl.ds(wid * _R * _D, _R * _D)])


_sc_seg_max = functools.partial(
    pl.kernel,
    out_type=jax.ShapeDtypeStruct((_NPAD * _D,), jnp.float32),
    mesh=plsc.VectorSubcoreMesh(core_axis_name="c", subcore_axis_name="s"),
    compiler_params=pltpu.CompilerParams(needs_layout_passes=False),
    scratch_types=[
        pltpu.VMEM((_R * _D,), jnp.float32),
        pltpu.VMEM((_CHUNK,), jnp.int32),
        pltpu.VMEM((_CHUNK,), jnp.int32),
        pltpu.VMEM((_CHUNK,), jnp.float32),
        pltpu.VMEM((_K + _L,), jnp.int32),
        pltpu.VMEM((_K + _L,), jnp.int32),
        pltpu.VMEM((_K + _L,), jnp.float32),
        pltpu.VMEM((_K, _D), jnp.float32),
        pltpu.SemaphoreType.DMA,
    ],
)(_sc_body)


def kernel(x, edge_index, edge_weight, W_pool, b_pool, W_self, b_self,
           W_neigh, b_neigh):
    src = edge_index[0]
    dst = edge_index[1]
    w = edge_weight[:, 0]

    h = pl.pallas_call(
        _mm_relu_body,
        grid=(_N // _BLK,),
        in_specs=[
            pl.BlockSpec((_BLK, _D), lambda i: (i, 0)),
            pl.BlockSpec((_D, _D), lambda i: (0, 0)),
            pl.BlockSpec((1, _D), lambda i: (0, 0)),
        ],
        out_specs=pl.BlockSpec((_BLK, _D), lambda i: (i, 0)),
        out_shape=jax.ShapeDtypeStruct((_N, _D), jnp.float32),
    )(x, W_pool.T, b_pool.reshape(1, _D))

    neigh = _sc_seg_max(h, src, dst, w).reshape(_NPAD, _D)[:_N]

    out = pl.pallas_call(
        _final_body,
        grid=(_N // _BLK,),
        in_specs=[
            pl.BlockSpec((_BLK, _D), lambda i: (i, 0)),
            pl.BlockSpec((_BLK, _D), lambda i: (i, 0)),
            pl.BlockSpec((_D, _D), lambda i: (0, 0)),
            pl.BlockSpec((_D, _D), lambda i: (0, 0)),
            pl.BlockSpec((1, _D), lambda i: (0, 0)),
        ],
        out_specs=pl.BlockSpec((_BLK, _D), lambda i: (i, 0)),
        out_shape=jax.ShapeDtypeStruct((_N, _D), jnp.float32),
    )(x, neigh, W_self.T, W_neigh.T, (b_self + b_neigh).reshape(1, _D))
    return out


# speedup vs baseline: 1.4185x; 1.0501x over previous
import functools

import jax
import jax.numpy as jnp
from jax import lax
from jax.experimental import pallas as pl
from jax.experimental.pallas import tpu as pltpu
from jax.experimental.pallas import tpu_sc as plsc

_N = 10000
_E = 320000
_D = 128

_NC = 2
_NS = 16
_NW = _NC * _NS
_L = 16

_R = 320
_NPAD = _NW * _R
_CHUNK = 4000
_NCHUNK = _E // _CHUNK
_K = 128
_FG = _D // _L

_BLK = 1000


def _mm_relu_body(x_ref, w_ref, b_ref, o_ref):
    acc = jnp.dot(x_ref[...], w_ref[...], preferred_element_type=jnp.float32)
    o_ref[...] = jnp.maximum(acc + b_ref[...], 0.0)


def _final_body(x_ref, n_ref, ws_ref, wn_ref, b_ref, o_ref):
    acc = jnp.dot(x_ref[...], ws_ref[...], preferred_element_type=jnp.float32)
    acc = acc + jnp.dot(n_ref[...], wn_ref[...], preferred_element_type=jnp.float32)
    o_ref[...] = acc + b_ref[...]


def _sc_body(h_hbm, src_hbm, dst_hbm, w_hbm, out_hbm,
             acc, dstc, srcc, wc, gidx, locb, wbuf, rows, sem):
    wid = lax.axis_index("s") * _NC + lax.axis_index("c")
    lo = wid * _R

    def _zero(i, _):
        acc[pl.ds(i * _L, _L)] = jnp.zeros((_L,), jnp.float32)
        return 0
    lax.fori_loop(0, (_R * _D) // _L, _zero, 0)
    for q in range((_K + _L) // _L):
        gidx[pl.ds(q * _L, _L)] = jnp.zeros((_L,), jnp.int32)
        locb[pl.ds(q * _L, _L)] = jnp.zeros((_L,), jnp.int32)

    def _flush(n):
        pltpu.async_copy(h_hbm.at[gidx.at[pl.ds(0, _K)]], rows, sem).wait()

        def _edge(i, _):
            loc = locb[pl.ds(i, _L)][0]
            wv = wbuf[pl.ds(i, _L)][0]
            base = loc * _D
            for f in range(_FG):
                a = acc[pl.ds(base + f * _L, _L)]
                r = rows[i, pl.ds(f * _L, _L)]
                acc[pl.ds(base + f * _L, _L)] = jnp.maximum(a, r * wv)
            return 0
        lax.fori_loop(0, n, _edge, 0)

    def _chunk(c, fill):
        cbase = c * _CHUNK
        pltpu.sync_copy(dst_hbm.at[pl.ds(cbase, _CHUNK)], dstc)
        pltpu.sync_copy(src_hbm.at[pl.ds(cbase, _CHUNK)], srcc)
        pltpu.sync_copy(w_hbm.at[pl.ds(cbase, _CHUNK)], wc)

        def _group(j, fill):
            d16 = dstc[pl.ds(j * _L, _L)]
            s16 = srcc[pl.ds(j * _L, _L)]
            w16 = wc[pl.ds(j * _L, _L)]
            m = (d16 >= lo) & (d16 < lo + _R)
            plsc.store_compressed(gidx.at[pl.ds(fill, _L)], s16, mask=m)
            plsc.store_compressed(locb.at[pl.ds(fill, _L)], d16 - lo, mask=m)
            plsc.store_compressed(wbuf.at[pl.ds(fill, _L)], w16, mask=m)
            fill = fill + plsc.all_reduce_population_count(m)[0]

            @pl.when(fill >= _K)
            def _():
                _flush(_K)
                gidx[pl.ds(0, _L)] = gidx[pl.ds(_K, _L)]
                locb[pl.ds(0, _L)] = locb[pl.ds(_K, _L)]
                wbuf[pl.ds(0, _L)] = wbuf[pl.ds(_K, _L)]
            return jnp.where(fill >= _K, fill - _K, fill)

        return lax.fori_loop(0, _CHUNK // _L, _group, fill)

    fill = lax.fori_loop(0, _NCHUNK, _chunk, jnp.int32(0))
    _flush(fill)
    pltpu.sync_copy(acc, out_hbm.at[pl.ds(wid * _R * _D, _R * _D)])


_sc_seg_max = functools.partial(
    pl.kernel,
    out_type=jax.ShapeDtypeStruct((_NPAD * _D,), jnp.float32),
    mesh=plsc.VectorSubcoreMesh(core_axis_name="c", subcore_axis_name="s"),
    compiler_params=pltpu.CompilerParams(needs_layout_passes=False),
    scratch_types=[
        pltpu.VMEM((_R * _D,), jnp.float32),
        pltpu.VMEM((_CHUNK,), jnp.int32),
        pltpu.VMEM((_CHUNK,), jnp.int32),
        pltpu.VMEM((_CHUNK,), jnp.float32),
        pltpu.VMEM((_K + _L,), jnp.int32),
        pltpu.VMEM((_K + _L,), jnp.int32),
        pltpu.VMEM((_K + _L,), jnp.float32),
        pltpu.VMEM((_K, _D), jnp.float32),
        pltpu.SemaphoreType.DMA,
    ],
)(_sc_body)


def kernel(x, edge_index, edge_weight, W_pool, b_pool, W_self, b_self,
           W_neigh, b_neigh):
    src = edge_index[0]
    dst = edge_index[1]
    w = edge_weight[:, 0]

    h = pl.pallas_call(
        _mm_relu_body,
        grid=(_N // _BLK,),
        in_specs=[
            pl.BlockSpec((_BLK, _D), lambda i: (i, 0)),
            pl.BlockSpec((_D, _D), lambda i: (0, 0)),
            pl.BlockSpec((1, _D), lambda i: (0, 0)),
        ],
        out_specs=pl.BlockSpec((_BLK, _D), lambda i: (i, 0)),
        out_shape=jax.ShapeDtypeStruct((_N, _D), jnp.float32),
    )(x, W_pool.T, b_pool.reshape(1, _D))

    neigh = _sc_seg_max(h, src, dst, w).reshape(_NPAD, _D)[:_N]

    out = pl.pallas_call(
        _final_body,
        grid=(_N // _BLK,),
        in_specs=[
            pl.BlockSpec((_BLK, _D), lambda i: (i, 0)),
            pl.BlockSpec((_BLK, _D), lambda i: (i, 0)),
            pl.BlockSpec((_D, _D), lambda i: (0, 0)),
            pl.BlockSpec((_D, _D), lambda i: (0, 0)),
            pl.BlockSpec((1, _D), lambda i: (0, 0)),
        ],
        out_specs=pl.BlockSpec((_BLK, _D), lambda i: (i, 0)),
        out_shape=jax.ShapeDtypeStruct((_N, _D), jnp.float32),
    )(x, neigh, W_self.T, W_neigh.T, (b_self + b_neigh).reshape(1, _D))
    return out


# speedup vs baseline: 2.0042x; 1.4129x over previous
import functools

import jax
import jax.numpy as jnp
from jax import lax
from jax.experimental import pallas as pl
from jax.experimental.pallas import tpu as pltpu
from jax.experimental.pallas import tpu_sc as plsc

_N = 10000
_E = 320000
_D = 128

_NC = 2
_NS = 16
_NW = _NC * _NS
_L = 16

_R = 320
_NPAD = _NW * _R
_CHUNK = 4000
_NCHUNK = _E // _CHUNK
_K = 128
_FG = _D // _L

_BLK = 1000


def _mm_relu_body(x_ref, w_ref, b_ref, o_ref):
    acc = jnp.dot(x_ref[...], w_ref[...], preferred_element_type=jnp.float32)
    o_ref[...] = jnp.maximum(acc + b_ref[...], 0.0)


def _final_body(x_ref, n_ref, ws_ref, wn_ref, b_ref, o_ref):
    acc = jnp.dot(x_ref[...], ws_ref[...], preferred_element_type=jnp.float32)
    acc = acc + jnp.dot(n_ref[...], wn_ref[...], preferred_element_type=jnp.float32)
    o_ref[...] = acc + b_ref[...]


def _sc_body(h_hbm, src_hbm, dst_hbm, w_hbm, out_hbm,
             acc, dstc, srcc, wc, gidx, locb, wbuf, rows, sem):
    wid = lax.axis_index("s") * _NC + lax.axis_index("c")
    lo = wid * _R

    def _zero(i, _):
        acc[pl.ds(i * _L, _L)] = jnp.zeros((_L,), jnp.float32)
        return 0
    lax.fori_loop(0, (_R * _D) // _L, _zero, 0)
    for q in range((_K + _L) // _L):
        gidx[pl.ds(q * _L, _L)] = jnp.zeros((_L,), jnp.int32)
        locb[pl.ds(q * _L, _L)] = jnp.zeros((_L,), jnp.int32)

    def _flush(n):
        pltpu.async_copy(h_hbm.at[gidx.at[pl.ds(0, _K)]], rows, sem).wait()

        def _edge(i, base):
            nxt = locb[pl.ds(i + 1, _L)][0] * _D
            wv = wbuf[pl.ds(i, _L)][0]
            avals = [acc[pl.ds(base + f * _L, _L)] for f in range(_FG)]
            rvals = [rows[i, pl.ds(f * _L, _L)] for f in range(_FG)]
            for f in range(_FG):
                acc[pl.ds(base + f * _L, _L)] = jnp.maximum(
                    avals[f], rvals[f] * wv)
            return nxt
        base0 = locb[pl.ds(0, _L)][0] * _D
        lax.fori_loop(0, n, _edge, base0)

    def _chunk(c, fill):
        cbase = c * _CHUNK
        pltpu.sync_copy(dst_hbm.at[pl.ds(cbase, _CHUNK)], dstc)
        pltpu.sync_copy(src_hbm.at[pl.ds(cbase, _CHUNK)], srcc)
        pltpu.sync_copy(w_hbm.at[pl.ds(cbase, _CHUNK)], wc)

        def _group(j, fill):
            d16 = dstc[pl.ds(j * _L, _L)]
            s16 = srcc[pl.ds(j * _L, _L)]
            w16 = wc[pl.ds(j * _L, _L)]
            m = (d16 >= lo) & (d16 < lo + _R)
            plsc.store_compressed(gidx.at[pl.ds(fill, _L)], s16, mask=m)
            plsc.store_compressed(locb.at[pl.ds(fill, _L)], d16 - lo, mask=m)
            plsc.store_compressed(wbuf.at[pl.ds(fill, _L)], w16, mask=m)
            fill = fill + plsc.all_reduce_population_count(m)[0]

            @pl.when(fill >= _K)
            def _():
                _flush(_K)
                gidx[pl.ds(0, _L)] = gidx[pl.ds(_K, _L)]
                locb[pl.ds(0, _L)] = locb[pl.ds(_K, _L)]
                wbuf[pl.ds(0, _L)] = wbuf[pl.ds(_K, _L)]
            return jnp.where(fill >= _K, fill - _K, fill)

        return lax.fori_loop(0, _CHUNK // _L, _group, fill)

    fill = lax.fori_loop(0, _NCHUNK, _chunk, jnp.int32(0))
    _flush(fill)
    pltpu.sync_copy(acc, out_hbm.at[pl.ds(wid * _R * _D, _R * _D)])


_sc_seg_max = functools.partial(
    pl.kernel,
    out_type=jax.ShapeDtypeStruct((_NPAD * _D,), jnp.float32),
    mesh=plsc.VectorSubcoreMesh(core_axis_name="c", subcore_axis_name="s"),
    compiler_params=pltpu.CompilerParams(needs_layout_passes=False),
    scratch_types=[
        pltpu.VMEM((_R * _D,), jnp.float32),
        pltpu.VMEM((_CHUNK,), jnp.int32),
        pltpu.VMEM((_CHUNK,), jnp.int32),
        pltpu.VMEM((_CHUNK,), jnp.float32),
        pltpu.VMEM((_K + _L,), jnp.int32),
        pltpu.VMEM((_K + _L,), jnp.int32),
        pltpu.VMEM((_K + _L,), jnp.float32),
        pltpu.VMEM((_K, _D), jnp.float32),
        pltpu.SemaphoreType.DMA,
    ],
)(_sc_body)


def kernel(x, edge_index, edge_weight, W_pool, b_pool, W_self, b_self,
           W_neigh, b_neigh):
    src = edge_index[0]
    dst = edge_index[1]
    w = edge_weight[:, 0]

    h = pl.pallas_call(
        _mm_relu_body,
        grid=(_N // _BLK,),
        in_specs=[
            pl.BlockSpec((_BLK, _D), lambda i: (i, 0)),
            pl.BlockSpec((_D, _D), lambda i: (0, 0)),
            pl.BlockSpec((1, _D), lambda i: (0, 0)),
        ],
        out_specs=pl.BlockSpec((_BLK, _D), lambda i: (i, 0)),
        out_shape=jax.ShapeDtypeStruct((_N, _D), jnp.float32),
    )(x, W_pool.T, b_pool.reshape(1, _D))

    neigh = _sc_seg_max(h, src, dst, w).reshape(_NPAD, _D)[:_N]

    out = pl.pallas_call(
        _final_body,
        grid=(_N // _BLK,),
        in_specs=[
            pl.BlockSpec((_BLK, _D), lambda i: (i, 0)),
            pl.BlockSpec((_BLK, _D), lambda i: (i, 0)),
            pl.BlockSpec((_D, _D), lambda i: (0, 0)),
            pl.BlockSpec((_D, _D), lambda i: (0, 0)),
            pl.BlockSpec((1, _D), lambda i: (0, 0)),
        ],
        out_specs=pl.BlockSpec((_BLK, _D), lambda i: (i, 0)),
        out_shape=jax.ShapeDtypeStruct((_N, _D), jnp.float32),
    )(x, neigh, W_self.T, W_neigh.T, (b_self + b_neigh).reshape(1, _D))
    return out


# speedup vs baseline: 2.7092x; 1.3518x over previous
import functools

import jax
import jax.numpy as jnp
from jax import lax
from jax.experimental import pallas as pl
from jax.experimental.pallas import tpu as pltpu
from jax.experimental.pallas import tpu_sc as plsc

_N = 10000
_E = 320000
_D = 128

_NC = 2
_NS = 16
_NW = _NC * _NS
_L = 16

_R = 320
_NPAD = _NW * _R
_CHUNK = 4000
_NCHUNK = _E // _CHUNK
_K = 128
_FG = _D // _L

_BLK = 1000


def _mm_relu_body(x_ref, w_ref, b_ref, o_ref):
    acc = jnp.dot(x_ref[...], w_ref[...], preferred_element_type=jnp.float32)
    o_ref[...] = jnp.maximum(acc + b_ref[...], 0.0)


def _final_body(x_ref, n_ref, ws_ref, wn_ref, b_ref, o_ref):
    acc = jnp.dot(x_ref[...], ws_ref[...], preferred_element_type=jnp.float32)
    acc = acc + jnp.dot(n_ref[...], wn_ref[...], preferred_element_type=jnp.float32)
    o_ref[...] = acc + b_ref[...]


def _sc_body(h_hbm, src_hbm, dst_hbm, w_hbm, out_hbm,
             acc, dstc, srcc, wc, gidx, locb, wbuf, rows, sem, sem0, sem1):
    wid = lax.axis_index("s") * _NC + lax.axis_index("c")
    lo = wid * _R

    def _zero(i, _):
        acc[pl.ds(i * _L, _L)] = jnp.zeros((_L,), jnp.float32)
        return 0
    lax.fori_loop(0, (_R * _D) // _L, _zero, 0)
    for q in range((_K + _L) // _L):
        gidx[pl.ds(q * _L, _L)] = jnp.zeros((_L,), jnp.int32)
        locb[pl.ds(q * _L, _L)] = jnp.zeros((_L,), jnp.int32)

    def _flush(n):
        pltpu.async_copy(h_hbm.at[gidx.at[pl.ds(0, _K)]], rows, sem).wait()

        def _edge(i, base):
            nxt = locb[pl.ds(i + 1, _L)][0] * _D
            wv = wbuf[pl.ds(i, _L)][0]
            avals = [acc[pl.ds(base + f * _L, _L)] for f in range(_FG)]
            rvals = [rows[i, pl.ds(f * _L, _L)] for f in range(_FG)]
            for f in range(_FG):
                acc[pl.ds(base + f * _L, _L)] = jnp.maximum(
                    avals[f], rvals[f] * wv)
            return nxt
        base0 = locb[pl.ds(0, _L)][0] * _D
        lax.fori_loop(0, n, _edge, base0)

    def _spill():
        _flush(_K)
        gidx[pl.ds(0, _L)] = gidx[pl.ds(_K, _L)]
        locb[pl.ds(0, _L)] = locb[pl.ds(_K, _L)]
        wbuf[pl.ds(0, _L)] = wbuf[pl.ds(_K, _L)]

    def _fire(c, sbase, s):
        cb = c * _CHUNK
        pltpu.async_copy(dst_hbm.at[pl.ds(cb, _CHUNK)],
                         dstc.at[pl.ds(sbase, _CHUNK)], s)
        pltpu.async_copy(src_hbm.at[pl.ds(cb, _CHUNK)],
                         srcc.at[pl.ds(sbase, _CHUNK)], s)
        pltpu.async_copy(w_hbm.at[pl.ds(cb, _CHUNK)],
                         wc.at[pl.ds(sbase, _CHUNK)], s)

    def _drain(sbase, s):
        pltpu.make_async_copy(dst_hbm.at[pl.ds(0, _CHUNK)],
                              dstc.at[pl.ds(sbase, _CHUNK)], s).wait()
        pltpu.make_async_copy(src_hbm.at[pl.ds(0, _CHUNK)],
                              srcc.at[pl.ds(sbase, _CHUNK)], s).wait()
        pltpu.make_async_copy(w_hbm.at[pl.ds(0, _CHUNK)],
                              wc.at[pl.ds(sbase, _CHUNK)], s).wait()

    def _scan(sbase, carry):
        def _group(j, carry):
            fill, cntp = carry
            fill = fill + cntp

            @pl.when(fill >= _K)
            def _():
                _spill()
            fill = jnp.where(fill >= _K, fill - _K, fill)

            d16 = dstc[pl.ds(sbase + j * _L, _L)]
            s16 = srcc[pl.ds(sbase + j * _L, _L)]
            w16 = wc[pl.ds(sbase + j * _L, _L)]
            m = (d16 >= lo) & (d16 < lo + _R)
            plsc.store_compressed(gidx.at[pl.ds(fill, _L)], s16, mask=m)
            plsc.store_compressed(locb.at[pl.ds(fill, _L)], d16 - lo, mask=m)
            plsc.store_compressed(wbuf.at[pl.ds(fill, _L)], w16, mask=m)
            cnt = plsc.all_reduce_population_count(m)[0]
            return fill, cnt

        return lax.fori_loop(0, _CHUNK // _L, _group, carry)

    def _pair(i, carry):
        _fire(2 * i + 1, _CHUNK, sem1)
        _drain(0, sem0)
        carry = _scan(0, carry)
        _fire(jnp.minimum(2 * i + 2, _NCHUNK - 1), 0, sem0)
        _drain(_CHUNK, sem1)
        return _scan(_CHUNK, carry)

    _fire(0, 0, sem0)
    fill, cntp = lax.fori_loop(0, _NCHUNK // 2, _pair,
                               (jnp.int32(0), jnp.int32(0)))
    _drain(0, sem0)
    fill = fill + cntp

    @pl.when(fill >= _K)
    def _():
        _spill()
    fill = jnp.where(fill >= _K, fill - _K, fill)
    _flush(fill)
    pltpu.sync_copy(acc, out_hbm.at[pl.ds(wid * _R * _D, _R * _D)])


_sc_seg_max = functools.partial(
    pl.kernel,
    out_type=jax.ShapeDtypeStruct((_NPAD * _D,), jnp.float32),
    mesh=plsc.VectorSubcoreMesh(core_axis_name="c", subcore_axis_name="s"),
    compiler_params=pltpu.CompilerParams(needs_layout_passes=False),
    scratch_types=[
        pltpu.VMEM((_R * _D,), jnp.float32),
        pltpu.VMEM((2 * _CHUNK,), jnp.int32),
        pltpu.VMEM((2 * _CHUNK,), jnp.int32),
        pltpu.VMEM((2 * _CHUNK,), jnp.float32),
        pltpu.VMEM((_K + _L,), jnp.int32),
        pltpu.VMEM((_K + _L,), jnp.int32),
        pltpu.VMEM((_K + _L,), jnp.float32),
        pltpu.VMEM((_K, _D), jnp.float32),
        pltpu.SemaphoreType.DMA,
        pltpu.SemaphoreType.DMA,
        pltpu.SemaphoreType.DMA,
    ],
)(_sc_body)


def kernel(x, edge_index, edge_weight, W_pool, b_pool, W_self, b_self,
           W_neigh, b_neigh):
    src = edge_index[0]
    dst = edge_index[1]
    w = edge_weight[:, 0]

    h = pl.pallas_call(
        _mm_relu_body,
        grid=(_N // _BLK,),
        in_specs=[
            pl.BlockSpec((_BLK, _D), lambda i: (i, 0)),
            pl.BlockSpec((_D, _D), lambda i: (0, 0)),
            pl.BlockSpec((1, _D), lambda i: (0, 0)),
        ],
        out_specs=pl.BlockSpec((_BLK, _D), lambda i: (i, 0)),
        out_shape=jax.ShapeDtypeStruct((_N, _D), jnp.float32),
    )(x, W_pool.T, b_pool.reshape(1, _D))

    neigh = _sc_seg_max(h, src, dst, w).reshape(_NPAD, _D)[:_N]

    out = pl.pallas_call(
        _final_body,
        grid=(_N // _BLK,),
        in_specs=[
            pl.BlockSpec((_BLK, _D), lambda i: (i, 0)),
            pl.BlockSpec((_BLK, _D), lambda i: (i, 0)),
            pl.BlockSpec((_D, _D), lambda i: (0, 0)),
            pl.BlockSpec((_D, _D), lambda i: (0, 0)),
            pl.BlockSpec((1, _D), lambda i: (0, 0)),
        ],
        out_specs=pl.BlockSpec((_BLK, _D), lambda i: (i, 0)),
        out_shape=jax.ShapeDtypeStruct((_N, _D), jnp.float32),
    )(x, neigh, W_self.T, W_neigh.T, (b_self + b_neigh).reshape(1, _D))
    return out


# speedup vs baseline: 3.2155x; 1.1869x over previous
import functools

import jax
import jax.numpy as jnp
from jax import lax
from jax.experimental import pallas as pl
from jax.experimental.pallas import tpu as pltpu
from jax.experimental.pallas import tpu_sc as plsc

_N = 10000
_E = 320000
_D = 128

_NC = 2
_NS = 16
_NW = _NC * _NS
_L = 16

_R = 320
_NPAD = _NW * _R
_CHUNK = 4000
_NCHUNK = _E // _CHUNK
_K = 128
_BUFS = _K + _L
_FG = _D // _L

_BLK = 1000


def _mm_relu_body(x_ref, w_ref, b_ref, o_ref):
    acc = jnp.dot(x_ref[...], w_ref[...], preferred_element_type=jnp.float32)
    o_ref[...] = jnp.maximum(acc + b_ref[...], 0.0)


def _final_body(x_ref, n_ref, ws_ref, wn_ref, b_ref, o_ref):
    acc = jnp.dot(x_ref[...], ws_ref[...], preferred_element_type=jnp.float32)
    acc = acc + jnp.dot(n_ref[...], wn_ref[...], preferred_element_type=jnp.float32)
    o_ref[...] = acc + b_ref[...]


def _sc_body(h_hbm, src_hbm, dst_hbm, w_hbm, out_hbm,
             acc, dstc, srcc, wc, gidx, locb, wbuf, rows, sem, sem0, sem1):
    wid = lax.axis_index("s") * _NC + lax.axis_index("c")
    lo = wid * _R

    def _zero(i, _):
        acc[pl.ds(i * _L, _L)] = jnp.zeros((_L,), jnp.float32)
        return 0
    lax.fori_loop(0, (_R * _D) // _L, _zero, 0)
    for q in range(2 * _BUFS // _L):
        gidx[pl.ds(q * _L, _L)] = jnp.zeros((_L,), jnp.int32)
        locb[pl.ds(q * _L, _L)] = jnp.zeros((_L,), jnp.int32)

    def _rmw(slot, n):
        sb = slot * _BUFS
        rb = slot * _K

        def _edge(i, base):
            nxt = locb[pl.ds(sb + i + 1, _L)][0] * _D
            wv = wbuf[pl.ds(sb + i, _L)][0]
            avals = [acc[pl.ds(base + f * _L, _L)] for f in range(_FG)]
            rvals = [rows[rb + i, pl.ds(f * _L, _L)] for f in range(_FG)]
            for f in range(_FG):
                acc[pl.ds(base + f * _L, _L)] = jnp.maximum(
                    avals[f], rvals[f] * wv)
            return nxt
        base0 = locb[pl.ds(sb, _L)][0] * _D
        lax.fori_loop(0, n, _edge, base0)

    def _fire_rows(slot):
        pltpu.async_copy(h_hbm.at[gidx.at[pl.ds(slot * _BUFS, _K)]],
                         rows.at[pl.ds(slot * _K, _K)], sem)

    def _wait_rows(slot):
        pltpu.make_async_copy(h_hbm.at[gidx.at[pl.ds(slot * _BUFS, _K)]],
                              rows.at[pl.ds(slot * _K, _K)], sem).wait()

    def _spill(fill, slot, pend):
        c = fill >= _K

        @pl.when(c)
        def _():
            @pl.when(pend == 1)
            def _():
                _wait_rows(1 - slot)
                _rmw(1 - slot, _K)
            _fire_rows(slot)
            osb = (1 - slot) * _BUFS
            sb = slot * _BUFS
            gidx[pl.ds(osb, _L)] = gidx[pl.ds(sb + _K, _L)]
            locb[pl.ds(osb, _L)] = locb[pl.ds(sb + _K, _L)]
            wbuf[pl.ds(osb, _L)] = wbuf[pl.ds(sb + _K, _L)]
        fill = jnp.where(c, fill - _K, fill)
        slot = jnp.where(c, 1 - slot, slot)
        pend = jnp.where(c, jnp.int32(1), pend)
        return fill, slot, pend

    def _fire(c, sbase, s):
        cb = c * _CHUNK
        pltpu.async_copy(dst_hbm.at[pl.ds(cb, _CHUNK)],
                         dstc.at[pl.ds(sbase, _CHUNK)], s)
        pltpu.async_copy(src_hbm.at[pl.ds(cb, _CHUNK)],
                         srcc.at[pl.ds(sbase, _CHUNK)], s)
        pltpu.async_copy(w_hbm.at[pl.ds(cb, _CHUNK)],
                         wc.at[pl.ds(sbase, _CHUNK)], s)

    def _drain(sbase, s):
        pltpu.make_async_copy(dst_hbm.at[pl.ds(0, _CHUNK)],
                              dstc.at[pl.ds(sbase, _CHUNK)], s).wait()
        pltpu.make_async_copy(src_hbm.at[pl.ds(0, _CHUNK)],
                              srcc.at[pl.ds(sbase, _CHUNK)], s).wait()
        pltpu.make_async_copy(w_hbm.at[pl.ds(0, _CHUNK)],
                              wc.at[pl.ds(sbase, _CHUNK)], s).wait()

    def _scan(sbase, carry):
        def _group(j, carry):
            fill, cntp, slot, pend = carry
            fill = fill + cntp
            fill, slot, pend = _spill(fill, slot, pend)

            sb = slot * _BUFS
            d16 = dstc[pl.ds(sbase + j * _L, _L)]
            s16 = srcc[pl.ds(sbase + j * _L, _L)]
            w16 = wc[pl.ds(sbase + j * _L, _L)]
            m = (d16 >= lo) & (d16 < lo + _R)
            plsc.store_compressed(gidx.at[pl.ds(sb + fill, _L)], s16, mask=m)
            plsc.store_compressed(locb.at[pl.ds(sb + fill, _L)], d16 - lo,
                                  mask=m)
            plsc.store_compressed(wbuf.at[pl.ds(sb + fill, _L)], w16, mask=m)
            cnt = plsc.all_reduce_population_count(m)[0]
            return fill, cnt, slot, pend

        return lax.fori_loop(0, _CHUNK // _L, _group, carry)

    def _pair(i, carry):
        _fire(2 * i + 1, _CHUNK, sem1)
        _drain(0, sem0)
        carry = _scan(0, carry)
        _fire(jnp.minimum(2 * i + 2, _NCHUNK - 1), 0, sem0)
        _drain(_CHUNK, sem1)
        return _scan(_CHUNK, carry)

    _fire(0, 0, sem0)
    fill, cntp, slot, pend = lax.fori_loop(
        0, _NCHUNK // 2, _pair,
        (jnp.int32(0), jnp.int32(0), jnp.int32(0), jnp.int32(0)))
    _drain(0, sem0)
    fill = fill + cntp
    fill, slot, pend = _spill(fill, slot, pend)

    @pl.when(pend == 1)
    def _():
        _wait_rows(1 - slot)
        _rmw(1 - slot, _K)
    _fire_rows(slot)
    _wait_rows(slot)
    _rmw(slot, fill)
    pltpu.sync_copy(acc, out_hbm.at[pl.ds(wid * _R * _D, _R * _D)])


_sc_seg_max = functools.partial(
    pl.kernel,
    out_type=jax.ShapeDtypeStruct((_NPAD * _D,), jnp.float32),
    mesh=plsc.VectorSubcoreMesh(core_axis_name="c", subcore_axis_name="s"),
    compiler_params=pltpu.CompilerParams(needs_layout_passes=False),
    scratch_types=[
        pltpu.VMEM((_R * _D,), jnp.float32),
        pltpu.VMEM((2 * _CHUNK,), jnp.int32),
        pltpu.VMEM((2 * _CHUNK,), jnp.int32),
        pltpu.VMEM((2 * _CHUNK,), jnp.float32),
        pltpu.VMEM((2 * _BUFS,), jnp.int32),
        pltpu.VMEM((2 * _BUFS,), jnp.int32),
        pltpu.VMEM((2 * _BUFS,), jnp.float32),
        pltpu.VMEM((2 * _K, _D), jnp.float32),
        pltpu.SemaphoreType.DMA,
        pltpu.SemaphoreType.DMA,
        pltpu.SemaphoreType.DMA,
    ],
)(_sc_body)


def kernel(x, edge_index, edge_weight, W_pool, b_pool, W_self, b_self,
           W_neigh, b_neigh):
    src = edge_index[0]
    dst = edge_index[1]
    w = edge_weight[:, 0]

    h = pl.pallas_call(
        _mm_relu_body,
        grid=(_N // _BLK,),
        in_specs=[
            pl.BlockSpec((_BLK, _D), lambda i: (i, 0)),
            pl.BlockSpec((_D, _D), lambda i: (0, 0)),
            pl.BlockSpec((1, _D), lambda i: (0, 0)),
        ],
        out_specs=pl.BlockSpec((_BLK, _D), lambda i: (i, 0)),
        out_shape=jax.ShapeDtypeStruct((_N, _D), jnp.float32),
    )(x, W_pool.T, b_pool.reshape(1, _D))

    neigh = _sc_seg_max(h, src, dst, w).reshape(_NPAD, _D)[:_N]

    out = pl.pallas_call(
        _final_body,
        grid=(_N // _BLK,),
        in_specs=[
            pl.BlockSpec((_BLK, _D), lambda i: (i, 0)),
            pl.BlockSpec((_BLK, _D), lambda i: (i, 0)),
            pl.BlockSpec((_D, _D), lambda i: (0, 0)),
            pl.BlockSpec((_D, _D), lambda i: (0, 0)),
            pl.BlockSpec((1, _D), lambda i: (0, 0)),
        ],
        out_specs=pl.BlockSpec((_BLK, _D), lambda i: (i, 0)),
        out_shape=jax.ShapeDtypeStruct((_N, _D), jnp.float32),
    )(x, neigh, W_self.T, W_neigh.T, (b_self + b_neigh).reshape(1, _D))
    return out


# speedup vs baseline: 4.0070x; 1.2462x over previous
import functools

import jax
import jax.numpy as jnp
from jax import lax
from jax.experimental import pallas as pl
from jax.experimental.pallas import tpu as pltpu
from jax.experimental.pallas import tpu_sc as plsc

_N = 10000
_E = 320000
_D = 128

_NC = 2
_NS = 16
_NW = _NC * _NS
_L = 16

_NRANGE = 16
_RP = 640
_NPAD = _NRANGE * _RP
_CHUNK = 2000
_NCHUNK = _E // _CHUNK
_NCH = _NCHUNK // 2
_K = 128
_BUFS = _K + _L
_FG = _D // _L
_MSL = 2048
_NMSL = (_RP * _D) // _MSL

_BLK = 1000


def _mm_relu_body(x_ref, w_ref, b_ref, o_ref):
    acc = jnp.dot(x_ref[...], w_ref[...], preferred_element_type=jnp.float32)
    o_ref[...] = jnp.maximum(acc + b_ref[...], 0.0)


def _final_body(x_ref, n_ref, ws_ref, wn_ref, b_ref, o_ref):
    acc = jnp.dot(x_ref[...], ws_ref[...], preferred_element_type=jnp.float32)
    acc = acc + jnp.dot(n_ref[...], wn_ref[...], preferred_element_type=jnp.float32)
    o_ref[...] = acc + b_ref[...]


def _sc_body(h_hbm, src_hbm, dst_hbm, w_hbm, out_hbm,
             acc, dstc, srcc, wc, gidx, locb, wbuf, rows, stag,
             sem, sem0, sem1):
    cid = lax.axis_index("c")
    sid = lax.axis_index("s")
    rng = cid * (_NS // 2) + sid // 2
    half = sid % 2
    lo = rng * _RP

    def _zero(i, _):
        acc[pl.ds(i * _L, _L)] = jnp.zeros((_L,), jnp.float32)
        return 0
    lax.fori_loop(0, (_RP * _D) // _L, _zero, 0)
    for q in range(2 * _BUFS // _L):
        gidx[pl.ds(q * _L, _L)] = jnp.zeros((_L,), jnp.int32)
        locb[pl.ds(q * _L, _L)] = jnp.zeros((_L,), jnp.int32)

    def _rmw(slot, n):
        sb = slot * _BUFS
        rb = slot * _K

        def _edge(i, base):
            nxt = locb[pl.ds(sb + i + 1, _L)][0] * _D
            wv = wbuf[pl.ds(sb + i, _L)][0]
            avals = [acc[pl.ds(base + f * _L, _L)] for f in range(_FG)]
            rvals = [rows[rb + i, pl.ds(f * _L, _L)] for f in range(_FG)]
            for f in range(_FG):
                acc[pl.ds(base + f * _L, _L)] = jnp.maximum(
                    avals[f], rvals[f] * wv)
            return nxt
        base0 = locb[pl.ds(sb, _L)][0] * _D
        lax.fori_loop(0, n, _edge, base0)

    def _fire_rows(slot):
        pltpu.async_copy(h_hbm.at[gidx.at[pl.ds(slot * _BUFS, _K)]],
                         rows.at[pl.ds(slot * _K, _K)], sem)

    def _wait_rows(slot):
        pltpu.make_async_copy(h_hbm.at[gidx.at[pl.ds(slot * _BUFS, _K)]],
                              rows.at[pl.ds(slot * _K, _K)], sem).wait()

    def _spill(fill, slot, pend):
        c = fill >= _K

        @pl.when(c)
        def _():
            @pl.when(pend == 1)
            def _():
                _wait_rows(1 - slot)
                _rmw(1 - slot, _K)
            _fire_rows(slot)
            osb = (1 - slot) * _BUFS
            sb = slot * _BUFS
            gidx[pl.ds(osb, _L)] = gidx[pl.ds(sb + _K, _L)]
            locb[pl.ds(osb, _L)] = locb[pl.ds(sb + _K, _L)]
            wbuf[pl.ds(osb, _L)] = wbuf[pl.ds(sb + _K, _L)]
        fill = jnp.where(c, fill - _K, fill)
        slot = jnp.where(c, 1 - slot, slot)
        pend = jnp.where(c, jnp.int32(1), pend)
        return fill, slot, pend

    def _fire(c, sbase, s):
        cb = c * _CHUNK
        pltpu.async_copy(dst_hbm.at[pl.ds(cb, _CHUNK)],
                         dstc.at[pl.ds(sbase, _CHUNK)], s)
        pltpu.async_copy(src_hbm.at[pl.ds(cb, _CHUNK)],
                         srcc.at[pl.ds(sbase, _CHUNK)], s)
        pltpu.async_copy(w_hbm.at[pl.ds(cb, _CHUNK)],
                         wc.at[pl.ds(sbase, _CHUNK)], s)

    def _drain(sbase, s):
        pltpu.make_async_copy(dst_hbm.at[pl.ds(0, _CHUNK)],
                              dstc.at[pl.ds(sbase, _CHUNK)], s).wait()
        pltpu.make_async_copy(src_hbm.at[pl.ds(0, _CHUNK)],
                              srcc.at[pl.ds(sbase, _CHUNK)], s).wait()
        pltpu.make_async_copy(w_hbm.at[pl.ds(0, _CHUNK)],
                              wc.at[pl.ds(sbase, _CHUNK)], s).wait()

    def _scan(sbase, carry):
        def _group(j, carry):
            fill, cntp, slot, pend = carry
            fill = fill + cntp
            fill, slot, pend = _spill(fill, slot, pend)

            sb = slot * _BUFS
            d16 = dstc[pl.ds(sbase + j * _L, _L)]
            s16 = srcc[pl.ds(sbase + j * _L, _L)]
            w16 = wc[pl.ds(sbase + j * _L, _L)]
            m = (d16 >= lo) & (d16 < lo + _RP)
            plsc.store_compressed(gidx.at[pl.ds(sb + fill, _L)], s16, mask=m)
            plsc.store_compressed(locb.at[pl.ds(sb + fill, _L)], d16 - lo,
                                  mask=m)
            plsc.store_compressed(wbuf.at[pl.ds(sb + fill, _L)], w16, mask=m)
            cnt = plsc.all_reduce_population_count(m)[0]
            return fill, cnt, slot, pend

        return lax.fori_loop(0, _CHUNK // _L, _group, carry)

    cbase0 = half * _NCH

    def _pair(i, carry):
        _fire(cbase0 + 2 * i + 1, _CHUNK, sem1)
        _drain(0, sem0)
        carry = _scan(0, carry)
        _fire(cbase0 + jnp.minimum(2 * i + 2, _NCH - 1), 0, sem0)
        _drain(_CHUNK, sem1)
        return _scan(_CHUNK, carry)

    _fire(cbase0, 0, sem0)
    fill, cntp, slot, pend = lax.fori_loop(
        0, _NCH // 2, _pair,
        (jnp.int32(0), jnp.int32(0), jnp.int32(0), jnp.int32(0)))
    _drain(0, sem0)
    fill = fill + cntp
    fill, slot, pend = _spill(fill, slot, pend)

    @pl.when(pend == 1)
    def _():
        _wait_rows(1 - slot)
        _rmw(1 - slot, _K)
    _fire_rows(slot)
    _wait_rows(slot)
    _rmw(slot, fill)

    obase = rng * _RP * _D

    @pl.when(half == 1)
    def _():
        pltpu.sync_copy(acc, out_hbm.at[pl.ds(obase, _RP * _D)])
    plsc.subcore_barrier()

    @pl.when(half == 0)
    def _():
        def _mslice(k, _):
            pltpu.sync_copy(out_hbm.at[pl.ds(obase + k * _MSL, _MSL)], stag)

            def _mrow(r, _):
                off = k * _MSL + r * _L
                acc[pl.ds(off, _L)] = jnp.maximum(acc[pl.ds(off, _L)],
                                                  stag[pl.ds(r * _L, _L)])
                return 0
            lax.fori_loop(0, _MSL // _L, _mrow, 0)
            return 0
        lax.fori_loop(0, _NMSL, _mslice, 0)
        pltpu.sync_copy(acc, out_hbm.at[pl.ds(obase, _RP * _D)])


_sc_seg_max = functools.partial(
    pl.kernel,
    out_type=jax.ShapeDtypeStruct((_NPAD * _D,), jnp.float32),
    mesh=plsc.VectorSubcoreMesh(core_axis_name="c", subcore_axis_name="s"),
    compiler_params=pltpu.CompilerParams(needs_layout_passes=False),
    scratch_types=[
        pltpu.VMEM((_RP * _D,), jnp.float32),
        pltpu.VMEM((2 * _CHUNK,), jnp.int32),
        pltpu.VMEM((2 * _CHUNK,), jnp.int32),
        pltpu.VMEM((2 * _CHUNK,), jnp.float32),
        pltpu.VMEM((2 * _BUFS,), jnp.int32),
        pltpu.VMEM((2 * _BUFS,), jnp.int32),
        pltpu.VMEM((2 * _BUFS,), jnp.float32),
        pltpu.VMEM((2 * _K, _D), jnp.float32),
        pltpu.VMEM((_MSL,), jnp.float32),
        pltpu.SemaphoreType.DMA,
        pltpu.SemaphoreType.DMA,
        pltpu.SemaphoreType.DMA,
    ],
)(_sc_body)


def kernel(x, edge_index, edge_weight, W_pool, b_pool, W_self, b_self,
           W_neigh, b_neigh):
    src = edge_index[0]
    dst = edge_index[1]
    w = edge_weight[:, 0]

    h = pl.pallas_call(
        _mm_relu_body,
        grid=(_N // _BLK,),
        in_specs=[
            pl.BlockSpec((_BLK, _D), lambda i: (i, 0)),
            pl.BlockSpec((_D, _D), lambda i: (0, 0)),
            pl.BlockSpec((1, _D), lambda i: (0, 0)),
        ],
        out_specs=pl.BlockSpec((_BLK, _D), lambda i: (i, 0)),
        out_shape=jax.ShapeDtypeStruct((_N, _D), jnp.float32),
    )(x, W_pool.T, b_pool.reshape(1, _D))

    neigh = _sc_seg_max(h, src, dst, w).reshape(_NPAD, _D)[:_N]

    out = pl.pallas_call(
        _final_body,
        grid=(_N // _BLK,),
        in_specs=[
            pl.BlockSpec((_BLK, _D), lambda i: (i, 0)),
            pl.BlockSpec((_BLK, _D), lambda i: (i, 0)),
            pl.BlockSpec((_D, _D), lambda i: (0, 0)),
            pl.BlockSpec((_D, _D), lambda i: (0, 0)),
            pl.BlockSpec((1, _D), lambda i: (0, 0)),
        ],
        out_specs=pl.BlockSpec((_BLK, _D), lambda i: (i, 0)),
        out_shape=jax.ShapeDtypeStruct((_N, _D), jnp.float32),
    )(x, neigh, W_self.T, W_neigh.T, (b_self + b_neigh).reshape(1, _D))
    return out


# speedup vs baseline: 4.2461x; 1.0597x over previous
import functools

import jax
import jax.numpy as jnp
from jax import lax
from jax.experimental import pallas as pl
from jax.experimental.pallas import tpu as pltpu
from jax.experimental.pallas import tpu_sc as plsc

_N = 10000
_E = 320000
_D = 128

_NC = 2
_NS = 16
_NW = _NC * _NS
_L = 16

_NRANGE = 16
_RP = 640
_NPAD = _NRANGE * _RP
_CHUNK = 4000
_NCHUNK = _E // _CHUNK
_NCH = _NCHUNK // 2
_K = 128
_BUFS = _K + _L
_L2 = 2 * _L
_FG2 = _D // _L2
_MSL = 2048
_NMSL = (_RP * _D) // _MSL

_BLK = 1000


def _mm_relu_body(x_ref, w_ref, b_ref, o_ref):
    acc = jnp.dot(x_ref[...], w_ref[...], preferred_element_type=jnp.float32)
    o_ref[...] = jnp.maximum(acc + b_ref[...], 0.0).astype(jnp.bfloat16)


def _final_body(x_ref, n_ref, ws_ref, wn_ref, b_ref, o_ref):
    acc = jnp.dot(x_ref[...], ws_ref[...], preferred_element_type=jnp.float32)
    acc = acc + jnp.dot(n_ref[...], wn_ref[...], preferred_element_type=jnp.float32)
    o_ref[...] = acc + b_ref[...]


def _sc_body(h_hbm, src_hbm, dst_hbm, w_hbm, out_hbm,
             acc, dstc, srcc, wc, gidx, locb, wbuf, rows, stag,
             sem, sem0, sem1):
    cid = lax.axis_index("c")
    sid = lax.axis_index("s")
    rng = cid * (_NS // 2) + sid // 2
    half = sid % 2
    lo = rng * _RP

    def _zero(i, _):
        acc[pl.ds(i * _L2, _L2)] = jnp.zeros((_L2,), jnp.bfloat16)
        return 0
    lax.fori_loop(0, (_RP * _D) // _L2, _zero, 0)
    for q in range(2 * _BUFS // _L):
        gidx[pl.ds(q * _L, _L)] = jnp.zeros((_L,), jnp.int32)
        locb[pl.ds(q * _L, _L)] = jnp.zeros((_L,), jnp.int32)

    def _rmw(slot, n):
        sb = slot * _BUFS
        rb = slot * _K

        def _edge(i, base):
            nxt = locb[pl.ds(sb + i + 1, _L)][0] * _D
            wv = wbuf[pl.ds(sb + i, _L)][0]
            wv2 = plsc.pack(jnp.full((_L,), wv, jnp.float32),
                            jnp.full((_L,), wv, jnp.float32),
                            format=plsc.PackFormat.INTERLEAVED)
            avals = [acc[pl.ds(base + f * _L2, _L2)] for f in range(_FG2)]
            rvals = [plsc.bitcast(rows[rb + i, pl.ds(f * _L, _L)],
                                  jnp.bfloat16) for f in range(_FG2)]
            for f in range(_FG2):
                acc[pl.ds(base + f * _L2, _L2)] = jnp.maximum(
                    avals[f], rvals[f] * wv2)
            return nxt
        base0 = locb[pl.ds(sb, _L)][0] * _D
        lax.fori_loop(0, n, _edge, base0)

    def _fire_rows(slot):
        pltpu.async_copy(h_hbm.at[gidx.at[pl.ds(slot * _BUFS, _K)]],
                         rows.at[pl.ds(slot * _K, _K)], sem)

    def _wait_rows(slot):
        pltpu.make_async_copy(h_hbm.at[gidx.at[pl.ds(slot * _BUFS, _K)]],
                              rows.at[pl.ds(slot * _K, _K)], sem).wait()

    def _spill(fill, slot, pend):
        c = fill >= _K

        @pl.when(c)
        def _():
            @pl.when(pend == 1)
            def _():
                _wait_rows(1 - slot)
                _rmw(1 - slot, _K)
            _fire_rows(slot)
            osb = (1 - slot) * _BUFS
            sb = slot * _BUFS
            gidx[pl.ds(osb, _L)] = gidx[pl.ds(sb + _K, _L)]
            locb[pl.ds(osb, _L)] = locb[pl.ds(sb + _K, _L)]
            wbuf[pl.ds(osb, _L)] = wbuf[pl.ds(sb + _K, _L)]
        fill = jnp.where(c, fill - _K, fill)
        slot = jnp.where(c, 1 - slot, slot)
        pend = jnp.where(c, jnp.int32(1), pend)
        return fill, slot, pend

    def _fire(c, sbase, s):
        cb = c * _CHUNK
        pltpu.async_copy(dst_hbm.at[pl.ds(cb, _CHUNK)],
                         dstc.at[pl.ds(sbase, _CHUNK)], s)
        pltpu.async_copy(src_hbm.at[pl.ds(cb, _CHUNK)],
                         srcc.at[pl.ds(sbase, _CHUNK)], s)
        pltpu.async_copy(w_hbm.at[pl.ds(cb, _CHUNK)],
                         wc.at[pl.ds(sbase, _CHUNK)], s)

    def _drain(sbase, s):
        pltpu.make_async_copy(dst_hbm.at[pl.ds(0, _CHUNK)],
                              dstc.at[pl.ds(sbase, _CHUNK)], s).wait()
        pltpu.make_async_copy(src_hbm.at[pl.ds(0, _CHUNK)],
                              srcc.at[pl.ds(sbase, _CHUNK)], s).wait()
        pltpu.make_async_copy(w_hbm.at[pl.ds(0, _CHUNK)],
                              wc.at[pl.ds(sbase, _CHUNK)], s).wait()

    def _scan(sbase, carry):
        def _group(j, carry):
            fill, cntp, slot, pend = carry
            fill = fill + cntp
            fill, slot, pend = _spill(fill, slot, pend)

            sb = slot * _BUFS
            d16 = dstc[pl.ds(sbase + j * _L, _L)]
            s16 = srcc[pl.ds(sbase + j * _L, _L)]
            w16 = wc[pl.ds(sbase + j * _L, _L)]
            m = (d16 >= lo) & (d16 < lo + _RP)
            plsc.store_compressed(gidx.at[pl.ds(sb + fill, _L)], s16, mask=m)
            plsc.store_compressed(locb.at[pl.ds(sb + fill, _L)], d16 - lo,
                                  mask=m)
            plsc.store_compressed(wbuf.at[pl.ds(sb + fill, _L)], w16, mask=m)
            cnt = plsc.all_reduce_population_count(m)[0]
            return fill, cnt, slot, pend

        return lax.fori_loop(0, _CHUNK // _L, _group, carry)

    cbase0 = half * _NCH

    def _pair(i, carry):
        _fire(cbase0 + 2 * i + 1, _CHUNK, sem1)
        _drain(0, sem0)
        carry = _scan(0, carry)
        _fire(cbase0 + jnp.minimum(2 * i + 2, _NCH - 1), 0, sem0)
        _drain(_CHUNK, sem1)
        return _scan(_CHUNK, carry)

    _fire(cbase0, 0, sem0)
    fill, cntp, slot, pend = lax.fori_loop(
        0, _NCH // 2, _pair,
        (jnp.int32(0), jnp.int32(0), jnp.int32(0), jnp.int32(0)))
    _drain(0, sem0)
    fill = fill + cntp
    fill, slot, pend = _spill(fill, slot, pend)

    @pl.when(pend == 1)
    def _():
        _wait_rows(1 - slot)
        _rmw(1 - slot, _K)
    _fire_rows(slot)
    _wait_rows(slot)
    _rmw(slot, fill)

    obase = rng * _RP * _D

    @pl.when(half == 1)
    def _():
        pltpu.sync_copy(acc, out_hbm.at[pl.ds(obase, _RP * _D)])
    plsc.subcore_barrier()

    @pl.when(half == 0)
    def _():
        def _mslice(k, _):
            pltpu.sync_copy(out_hbm.at[pl.ds(obase + k * _MSL, _MSL)], stag)

            def _mrow(r, _):
                off = k * _MSL + r * _L2
                acc[pl.ds(off, _L2)] = jnp.maximum(acc[pl.ds(off, _L2)],
                                                   stag[pl.ds(r * _L2, _L2)])
                return 0
            lax.fori_loop(0, _MSL // _L2, _mrow, 0)
            return 0
        lax.fori_loop(0, _NMSL, _mslice, 0)
        pltpu.sync_copy(acc, out_hbm.at[pl.ds(obase, _RP * _D)])


_sc_seg_max = functools.partial(
    pl.kernel,
    out_type=jax.ShapeDtypeStruct((_NPAD * _D,), jnp.bfloat16),
    mesh=plsc.VectorSubcoreMesh(core_axis_name="c", subcore_axis_name="s"),
    compiler_params=pltpu.CompilerParams(needs_layout_passes=False,
                                         use_tc_tiling_on_sc=False),
    scratch_types=[
        pltpu.VMEM((_RP * _D,), jnp.bfloat16),
        pltpu.VMEM((2 * _CHUNK,), jnp.int32),
        pltpu.VMEM((2 * _CHUNK,), jnp.int32),
        pltpu.VMEM((2 * _CHUNK,), jnp.float32),
        pltpu.VMEM((2 * _BUFS,), jnp.int32),
        pltpu.VMEM((2 * _BUFS,), jnp.int32),
        pltpu.VMEM((2 * _BUFS,), jnp.float32),
        pltpu.VMEM((2 * _K, _D // 2), jnp.int32),
        pltpu.VMEM((_MSL,), jnp.bfloat16),
        pltpu.SemaphoreType.DMA,
        pltpu.SemaphoreType.DMA,
        pltpu.SemaphoreType.DMA,
    ],
)(_sc_body)


def kernel(x, edge_index, edge_weight, W_pool, b_pool, W_self, b_self,
           W_neigh, b_neigh):
    src = edge_index[0]
    dst = edge_index[1]
    w = edge_weight[:, 0]

    h = pl.pallas_call(
        _mm_relu_body,
        grid=(_N // _BLK,),
        in_specs=[
            pl.BlockSpec((_BLK, _D), lambda i: (i, 0)),
            pl.BlockSpec((_D, _D), lambda i: (0, 0)),
            pl.BlockSpec((1, _D), lambda i: (0, 0)),
        ],
        out_specs=pl.BlockSpec((_BLK, _D), lambda i: (i, 0)),
        out_shape=jax.ShapeDtypeStruct((_N, _D), jnp.bfloat16),
    )(x, W_pool.T, b_pool.reshape(1, _D))

    h32 = lax.bitcast_convert_type(h.reshape(_N, _D // 2, 2),
                                   jnp.int32)
    neigh = (_sc_seg_max(h32, src, dst, w)
             .astype(jnp.float32).reshape(_NPAD, _D)[:_N])

    out = pl.pallas_call(
        _final_body,
        grid=(_N // _BLK,),
        in_specs=[
            pl.BlockSpec((_BLK, _D), lambda i: (i, 0)),
            pl.BlockSpec((_BLK, _D), lambda i: (i, 0)),
            pl.BlockSpec((_D, _D), lambda i: (0, 0)),
            pl.BlockSpec((_D, _D), lambda i: (0, 0)),
            pl.BlockSpec((1, _D), lambda i: (0, 0)),
        ],
        out_specs=pl.BlockSpec((_BLK, _D), lambda i: (i, 0)),
        out_shape=jax.ShapeDtypeStruct((_N, _D), jnp.float32),
    )(x, neigh, W_self.T, W_neigh.T, (b_self + b_neigh).reshape(1, _D))
    return out


# speedup vs baseline: 4.4851x; 1.0563x over previous
import functools

import jax
import jax.numpy as jnp
from jax import lax
from jax.experimental import pallas as pl
from jax.experimental.pallas import tpu as pltpu
from jax.experimental.pallas import tpu_sc as plsc

_N = 10000
_E = 320000
_D = 128

_NC = 2
_NS = 16
_NW = _NC * _NS
_L = 16

_NRANGE = 16
_RP = 640
_NPAD = _NRANGE * _RP
_CHUNK = 4000
_NCHUNK = _E // _CHUNK
_NCH = _NCHUNK // 2
_K = 128
_BUFS = _K + _L
_L2 = 2 * _L
_FG2 = _D // _L2
_MSL = 2048
_NMSL = (_RP * _D) // _MSL

_BLK = 1000


def _mm_relu_body(x_ref, w_ref, b_ref, o_ref):
    acc = jnp.dot(x_ref[...], w_ref[...], preferred_element_type=jnp.float32)
    o_ref[...] = jnp.maximum(acc + b_ref[...], 0.0).astype(jnp.bfloat16)


def _final_body(x_ref, n_ref, ws_ref, wn_ref, b_ref, o_ref):
    acc = jnp.dot(x_ref[...], ws_ref[...], preferred_element_type=jnp.float32)
    acc = acc + jnp.dot(n_ref[...], wn_ref[...], preferred_element_type=jnp.float32)
    o_ref[...] = acc + b_ref[...]


def _sc_body(h_hbm, src_hbm, dst_hbm, w_hbm, out_hbm,
             acc, dstc, srcc, wc, gidx, locb, wbuf, rows, stag,
             semA, semB, sem0, sem1):
    cid = lax.axis_index("c")
    sid = lax.axis_index("s")
    rng = cid * (_NS // 2) + sid // 2
    half = sid % 2
    lo = rng * _RP

    def _zero(i, _):
        acc[pl.ds(i * _L2, _L2)] = jnp.zeros((_L2,), jnp.bfloat16)
        return 0
    lax.fori_loop(0, (_RP * _D) // _L2, _zero, 0)
    for q in range(4 * _BUFS // _L):
        gidx[pl.ds(q * _L, _L)] = jnp.zeros((_L,), jnp.int32)
        locb[pl.ds(q * _L, _L)] = jnp.zeros((_L,), jnp.int32)

    def _rmw(slot, n):
        sb = slot * _BUFS
        rb = slot * _K

        def _edge(i, base):
            nxt = locb[pl.ds(sb + i + 1, _L)][0] * _D
            wv = wbuf[pl.ds(sb + i, _L)][0]
            wv2 = plsc.pack(jnp.full((_L,), wv, jnp.float32),
                            jnp.full((_L,), wv, jnp.float32),
                            format=plsc.PackFormat.INTERLEAVED)
            avals = [acc[pl.ds(base + f * _L2, _L2)] for f in range(_FG2)]
            rvals = [plsc.bitcast(rows[rb + i, pl.ds(f * _L, _L)],
                                  jnp.bfloat16) for f in range(_FG2)]
            for f in range(_FG2):
                acc[pl.ds(base + f * _L2, _L2)] = jnp.maximum(
                    avals[f], rvals[f] * wv2)
            return nxt
        base0 = locb[pl.ds(sb, _L)][0] * _D
        lax.fori_loop(0, n, _edge, base0)

    def _fire_rows(slot, sg):
        pltpu.async_copy(h_hbm.at[gidx.at[pl.ds(slot * _BUFS, _K)]],
                         rows.at[pl.ds(slot * _K, _K)], sg)

    def _wait_rows(slot, sg):
        pltpu.make_async_copy(h_hbm.at[gidx.at[pl.ds(slot * _BUFS, _K)]],
                              rows.at[pl.ds(slot * _K, _K)], sg).wait()

    def _spill(base, fill, slot, pend, sg):
        c = fill >= _K

        @pl.when(c)
        def _():
            cur = base + slot
            oth = base + 1 - slot

            @pl.when(pend == 1)
            def _():
                _wait_rows(oth, sg)
                _rmw(oth, _K)
            _fire_rows(cur, sg)
            osb = oth * _BUFS
            sb = cur * _BUFS
            gidx[pl.ds(osb, _L)] = gidx[pl.ds(sb + _K, _L)]
            locb[pl.ds(osb, _L)] = locb[pl.ds(sb + _K, _L)]
            wbuf[pl.ds(osb, _L)] = wbuf[pl.ds(sb + _K, _L)]
        fill = jnp.where(c, fill - _K, fill)
        slot = jnp.where(c, 1 - slot, slot)
        pend = jnp.where(c, jnp.int32(1), pend)
        return fill, slot, pend

    def _fire(c, sbase, s):
        cb = c * _CHUNK
        pltpu.async_copy(dst_hbm.at[pl.ds(cb, _CHUNK)],
                         dstc.at[pl.ds(sbase, _CHUNK)], s)
        pltpu.async_copy(src_hbm.at[pl.ds(cb, _CHUNK)],
                         srcc.at[pl.ds(sbase, _CHUNK)], s)
        pltpu.async_copy(w_hbm.at[pl.ds(cb, _CHUNK)],
                         wc.at[pl.ds(sbase, _CHUNK)], s)

    def _drain(sbase, s):
        pltpu.make_async_copy(dst_hbm.at[pl.ds(0, _CHUNK)],
                              dstc.at[pl.ds(sbase, _CHUNK)], s).wait()
        pltpu.make_async_copy(src_hbm.at[pl.ds(0, _CHUNK)],
                              srcc.at[pl.ds(sbase, _CHUNK)], s).wait()
        pltpu.make_async_copy(w_hbm.at[pl.ds(0, _CHUNK)],
                              wc.at[pl.ds(sbase, _CHUNK)], s).wait()

    def _step(sbase, g, base, sg, st):
        fill, cntp, slot, pend = st
        fill = fill + cntp
        fill, slot, pend = _spill(base, fill, slot, pend, sg)

        sb = (base + slot) * _BUFS
        d16 = dstc[pl.ds(sbase + g * _L, _L)]
        s16 = srcc[pl.ds(sbase + g * _L, _L)]
        w16 = wc[pl.ds(sbase + g * _L, _L)]
        m = (d16 >= lo) & (d16 < lo + _RP)
        plsc.store_compressed(gidx.at[pl.ds(sb + fill, _L)], s16, mask=m)
        plsc.store_compressed(locb.at[pl.ds(sb + fill, _L)], d16 - lo,
                              mask=m)
        plsc.store_compressed(wbuf.at[pl.ds(sb + fill, _L)], w16, mask=m)
        cnt = plsc.all_reduce_population_count(m)[0]
        return fill, cnt, slot, pend

    def _scan(sbase, carry):
        def _group(j, carry):
            stA, stB = carry
            stA = _step(sbase, 2 * j, 0, semA, stA)
            stB = _step(sbase, 2 * j + 1, 2, semB, stB)
            return stA, stB

        return lax.fori_loop(0, _CHUNK // (2 * _L), _group, carry)

    cbase0 = half * _NCH

    def _pair(i, carry):
        _fire(cbase0 + 2 * i + 1, _CHUNK, sem1)
        _drain(0, sem0)
        carry = _scan(0, carry)
        _fire(cbase0 + jnp.minimum(2 * i + 2, _NCH - 1), 0, sem0)
        _drain(_CHUNK, sem1)
        return _scan(_CHUNK, carry)

    _fire(cbase0, 0, sem0)
    st0 = (jnp.int32(0), jnp.int32(0), jnp.int32(0), jnp.int32(0))
    stA, stB = lax.fori_loop(0, _NCH // 2, _pair, (st0, st0))
    _drain(0, sem0)

    for base, (fill, cntp, slot, pend), sg in ((0, stA, semA), (2, stB, semB)):
        fill = fill + cntp
        fill, slot, pend = _spill(base, fill, slot, pend, sg)

        @pl.when(pend == 1)
        def _():
            _wait_rows(base + 1 - slot, sg)
            _rmw(base + 1 - slot, _K)
        _fire_rows(base + slot, sg)
        _wait_rows(base + slot, sg)
        _rmw(base + slot, fill)

    obase = rng * _RP * _D

    @pl.when(half == 1)
    def _():
        pltpu.sync_copy(acc, out_hbm.at[pl.ds(obase, _RP * _D)])
    plsc.subcore_barrier()

    @pl.when(half == 0)
    def _():
        def _mslice(k, _):
            pltpu.sync_copy(out_hbm.at[pl.ds(obase + k * _MSL, _MSL)], stag)

            def _mrow(r, _):
                off = k * _MSL + r * _L2
                acc[pl.ds(off, _L2)] = jnp.maximum(acc[pl.ds(off, _L2)],
                                                   stag[pl.ds(r * _L2, _L2)])
                return 0
            lax.fori_loop(0, _MSL // _L2, _mrow, 0)
            return 0
        lax.fori_loop(0, _NMSL, _mslice, 0)
        pltpu.sync_copy(acc, out_hbm.at[pl.ds(obase, _RP * _D)])


_sc_seg_max = functools.partial(
    pl.kernel,
    out_type=jax.ShapeDtypeStruct((_NPAD * _D,), jnp.bfloat16),
    mesh=plsc.VectorSubcoreMesh(core_axis_name="c", subcore_axis_name="s"),
    compiler_params=pltpu.CompilerParams(needs_layout_passes=False,
                                         use_tc_tiling_on_sc=False),
    scratch_types=[
        pltpu.VMEM((_RP * _D,), jnp.bfloat16),
        pltpu.VMEM((2 * _CHUNK,), jnp.int32),
        pltpu.VMEM((2 * _CHUNK,), jnp.int32),
        pltpu.VMEM((2 * _CHUNK,), jnp.float32),
        pltpu.VMEM((4 * _BUFS,), jnp.int32),
        pltpu.VMEM((4 * _BUFS,), jnp.int32),
        pltpu.VMEM((4 * _BUFS,), jnp.float32),
        pltpu.VMEM((4 * _K, _D // 2), jnp.int32),
        pltpu.VMEM((_MSL,), jnp.bfloat16),
        pltpu.SemaphoreType.DMA,
        pltpu.SemaphoreType.DMA,
        pltpu.SemaphoreType.DMA,
        pltpu.SemaphoreType.DMA,
    ],
)(_sc_body)


def kernel(x, edge_index, edge_weight, W_pool, b_pool, W_self, b_self,
           W_neigh, b_neigh):
    src = edge_index[0]
    dst = edge_index[1]
    w = edge_weight[:, 0]

    h = pl.pallas_call(
        _mm_relu_body,
        grid=(_N // _BLK,),
        in_specs=[
            pl.BlockSpec((_BLK, _D), lambda i: (i, 0)),
            pl.BlockSpec((_D, _D), lambda i: (0, 0)),
            pl.BlockSpec((1, _D), lambda i: (0, 0)),
        ],
        out_specs=pl.BlockSpec((_BLK, _D), lambda i: (i, 0)),
        out_shape=jax.ShapeDtypeStruct((_N, _D), jnp.bfloat16),
    )(x, W_pool.T, b_pool.reshape(1, _D))

    h32 = lax.bitcast_convert_type(h.reshape(_N, _D // 2, 2),
                                   jnp.int32)
    neigh = (_sc_seg_max(h32, src, dst, w)
             .astype(jnp.float32).reshape(_NPAD, _D)[:_N])

    out = pl.pallas_call(
        _final_body,
        grid=(_N // _BLK,),
        in_specs=[
            pl.BlockSpec((_BLK, _D), lambda i: (i, 0)),
            pl.BlockSpec((_BLK, _D), lambda i: (i, 0)),
            pl.BlockSpec((_D, _D), lambda i: (0, 0)),
            pl.BlockSpec((_D, _D), lambda i: (0, 0)),
            pl.BlockSpec((1, _D), lambda i: (0, 0)),
        ],
        out_specs=pl.BlockSpec((_BLK, _D), lambda i: (i, 0)),
        out_shape=jax.ShapeDtypeStruct((_N, _D), jnp.float32),
    )(x, neigh, W_self.T, W_neigh.T, (b_self + b_neigh).reshape(1, _D))
    return out


# speedup vs baseline: 4.4951x; 1.0022x over previous
import functools

import jax
import jax.numpy as jnp
from jax import lax
from jax.experimental import pallas as pl
from jax.experimental.pallas import tpu as pltpu
from jax.experimental.pallas import tpu_sc as plsc

_N = 10000
_E = 320000
_D = 128

_NC = 2
_NS = 16
_NW = _NC * _NS
_L = 16

_NRANGE = 16
_RP = 640
_NPAD = _NRANGE * _RP
_CHUNK = 4000
_NCHUNK = _E // _CHUNK
_NCH = _NCHUNK // 2
_K = 128
_BUFS = _K + _L
_L2 = 2 * _L
_FG2 = _D // _L2
_MSL = 2048
_NMSL = (_RP * _D) // _MSL

_BLK = 1000


def _mm_relu_body(x_ref, w_ref, b_ref, o_ref):
    acc = jnp.dot(x_ref[...], w_ref[...], preferred_element_type=jnp.float32)
    o_ref[...] = jnp.maximum(acc + b_ref[...], 0.0).astype(jnp.bfloat16)


def _final_body(x_ref, n_ref, ws_ref, wn_ref, b_ref, o_ref):
    acc = jnp.dot(x_ref[...], ws_ref[...], preferred_element_type=jnp.float32)
    acc = acc + jnp.dot(n_ref[...], wn_ref[...], preferred_element_type=jnp.float32)
    o_ref[...] = acc + b_ref[...]


def _sc_body(h_hbm, src_hbm, dst_hbm, w_hbm, out_hbm,
             acc, dstc, srcc, wc, gidx, locb, wbuf, rows, stag,
             semA, semB, sem0, sem1):
    cid = lax.axis_index("c")
    sid = lax.axis_index("s")
    rng = cid * (_NS // 2) + sid // 2
    half = sid % 2
    lo = rng * _RP

    def _zero(i, _):
        acc[pl.ds(i * _L2, _L2)] = jnp.zeros((_L2,), jnp.bfloat16)
        return 0
    lax.fori_loop(0, (_RP * _D) // _L2, _zero, 0)
    for q in range(4 * _BUFS // _L):
        gidx[pl.ds(q * _L, _L)] = jnp.zeros((_L,), jnp.int32)
        locb[pl.ds(q * _L, _L)] = jnp.zeros((_L,), jnp.int32)

    def _rmw(slot, n):
        sb = slot * _BUFS
        rb = slot * _K

        def _edge(i, base):
            nxt = locb[pl.ds(sb + i + 1, _L)][0] * _D
            wv = wbuf[pl.ds(sb + i, _L)][0]
            wv2 = plsc.pack(jnp.full((_L,), wv, jnp.float32),
                            jnp.full((_L,), wv, jnp.float32),
                            format=plsc.PackFormat.INTERLEAVED)
            avals = [acc[pl.ds(base + f * _L2, _L2)] for f in range(_FG2)]
            rvals = [plsc.bitcast(rows[rb + i, pl.ds(f * _L, _L)],
                                  jnp.bfloat16) for f in range(_FG2)]
            for f in range(_FG2):
                acc[pl.ds(base + f * _L2, _L2)] = jnp.maximum(
                    avals[f], rvals[f] * wv2)
            return nxt
        base0 = locb[pl.ds(sb, _L)][0] * _D
        lax.fori_loop(0, n, _edge, base0)

    def _fire_rows(slot, sg):
        pltpu.async_copy(h_hbm.at[gidx.at[pl.ds(slot * _BUFS, _K)]],
                         rows.at[pl.ds(slot * _K, _K)], sg)

    def _wait_rows(slot, sg):
        pltpu.make_async_copy(h_hbm.at[gidx.at[pl.ds(slot * _BUFS, _K)]],
                              rows.at[pl.ds(slot * _K, _K)], sg).wait()

    def _spill(base, fill, slot, pend, sg):
        c = fill >= _K

        @pl.when(c)
        def _():
            cur = base + slot
            oth = base + 1 - slot

            @pl.when(pend == 1)
            def _():
                _wait_rows(oth, sg)
                _rmw(oth, _K)
            _fire_rows(cur, sg)
            osb = oth * _BUFS
            sb = cur * _BUFS
            gidx[pl.ds(osb, _L)] = gidx[pl.ds(sb + _K, _L)]
            locb[pl.ds(osb, _L)] = locb[pl.ds(sb + _K, _L)]
            wbuf[pl.ds(osb, _L)] = wbuf[pl.ds(sb + _K, _L)]
        fill = jnp.where(c, fill - _K, fill)
        slot = jnp.where(c, 1 - slot, slot)
        pend = jnp.where(c, jnp.int32(1), pend)
        return fill, slot, pend

    def _fire(c, sbase, s):
        cb = c * _CHUNK
        pltpu.async_copy(dst_hbm.at[pl.ds(cb, _CHUNK)],
                         dstc.at[pl.ds(sbase, _CHUNK)], s)
        pltpu.async_copy(src_hbm.at[pl.ds(cb, _CHUNK)],
                         srcc.at[pl.ds(sbase, _CHUNK)], s)
        pltpu.async_copy(w_hbm.at[pl.ds(cb, _CHUNK)],
                         wc.at[pl.ds(sbase, _CHUNK)], s)

    def _drain(sbase, s):
        pltpu.make_async_copy(dst_hbm.at[pl.ds(0, _CHUNK)],
                              dstc.at[pl.ds(sbase, _CHUNK)], s).wait()
        pltpu.make_async_copy(src_hbm.at[pl.ds(0, _CHUNK)],
                              srcc.at[pl.ds(sbase, _CHUNK)], s).wait()
        pltpu.make_async_copy(w_hbm.at[pl.ds(0, _CHUNK)],
                              wc.at[pl.ds(sbase, _CHUNK)], s).wait()

    def _step(sbase, g, base, sg, st):
        fill, cntp, slot, pend = st
        fill = fill + cntp
        fill, slot, pend = _spill(base, fill, slot, pend, sg)

        sb = (base + slot) * _BUFS
        d16 = dstc[pl.ds(sbase + g * _L, _L)]
        s16 = srcc[pl.ds(sbase + g * _L, _L)]
        w16 = wc[pl.ds(sbase + g * _L, _L)]
        m = (d16 >= lo) & (d16 < lo + _RP)
        plsc.store_compressed(gidx.at[pl.ds(sb + fill, _L)], s16, mask=m)
        plsc.store_compressed(locb.at[pl.ds(sb + fill, _L)], d16 - lo,
                              mask=m)
        plsc.store_compressed(wbuf.at[pl.ds(sb + fill, _L)], w16, mask=m)
        cnt = plsc.all_reduce_population_count(m)[0]
        return fill, cnt, slot, pend

    def _scan(sbase, carry):
        def _group(j, carry):
            stA, stB = carry
            stA = _step(sbase, 2 * j, 0, semA, stA)
            stB = _step(sbase, 2 * j + 1, 2, semB, stB)
            return stA, stB

        return lax.fori_loop(0, _CHUNK // (2 * _L), _group, carry)

    cbase0 = half * _NCH

    def _pair(i, carry):
        _fire(cbase0 + 2 * i + 1, _CHUNK, sem1)
        _drain(0, sem0)
        carry = _scan(0, carry)
        _fire(cbase0 + jnp.minimum(2 * i + 2, _NCH - 1), 0, sem0)
        _drain(_CHUNK, sem1)
        return _scan(_CHUNK, carry)

    _fire(cbase0, 0, sem0)
    st0 = (jnp.int32(0), jnp.int32(0), jnp.int32(0), jnp.int32(0))
    stA, stB = lax.fori_loop(0, _NCH // 2, _pair, (st0, st0))
    _drain(0, sem0)

    for base, (fill, cntp, slot, pend), sg in ((0, stA, semA), (2, stB, semB)):
        fill = fill + cntp
        fill, slot, pend = _spill(base, fill, slot, pend, sg)

        @pl.when(pend == 1)
        def _():
            _wait_rows(base + 1 - slot, sg)
            _rmw(base + 1 - slot, _K)
        _fire_rows(base + slot, sg)
        _wait_rows(base + slot, sg)
        _rmw(base + slot, fill)

    obase = rng * _RP * _D

    @pl.when(half == 1)
    def _():
        pltpu.sync_copy(acc, out_hbm.at[pl.ds(obase, _RP * _D)])
    plsc.subcore_barrier()

    @pl.when(half == 0)
    def _():
        def _mslice(k, _):
            pltpu.sync_copy(out_hbm.at[pl.ds(obase + k * _MSL, _MSL)], stag)

            def _mrow(r, _):
                off = k * _MSL + r * _L2
                acc[pl.ds(off, _L2)] = jnp.maximum(acc[pl.ds(off, _L2)],
                                                   stag[pl.ds(r * _L2, _L2)])
                return 0
            lax.fori_loop(0, _MSL // _L2, _mrow, 0)
            return 0
        lax.fori_loop(0, _NMSL, _mslice, 0)
        pltpu.sync_copy(acc, out_hbm.at[pl.ds(obase, _RP * _D)])


_sc_seg_max = functools.partial(
    pl.kernel,
    out_type=jax.ShapeDtypeStruct((_NPAD * _D,), jnp.bfloat16),
    mesh=plsc.VectorSubcoreMesh(core_axis_name="c", subcore_axis_name="s"),
    compiler_params=pltpu.CompilerParams(needs_layout_passes=False,
                                         use_tc_tiling_on_sc=False),
    scratch_types=[
        pltpu.VMEM((_RP * _D,), jnp.bfloat16),
        pltpu.VMEM((2 * _CHUNK,), jnp.int32),
        pltpu.VMEM((2 * _CHUNK,), jnp.int32),
        pltpu.VMEM((2 * _CHUNK,), jnp.float32),
        pltpu.VMEM((4 * _BUFS,), jnp.int32),
        pltpu.VMEM((4 * _BUFS,), jnp.int32),
        pltpu.VMEM((4 * _BUFS,), jnp.float32),
        pltpu.VMEM((4 * _K, _D // 2), jnp.int32),
        pltpu.VMEM((_MSL,), jnp.bfloat16),
        pltpu.SemaphoreType.DMA,
        pltpu.SemaphoreType.DMA,
        pltpu.SemaphoreType.DMA,
        pltpu.SemaphoreType.DMA,
    ],
)(_sc_body)


def kernel(x, edge_index, edge_weight, W_pool, b_pool, W_self, b_self,
           W_neigh, b_neigh):
    src = edge_index[0]
    dst = edge_index[1]
    w = edge_weight[:, 0]

    h = pl.pallas_call(
        _mm_relu_body,
        grid=(_N // _BLK,),
        in_specs=[
            pl.BlockSpec((_BLK, _D), lambda i: (i, 0)),
            pl.BlockSpec((_D, _D), lambda i: (0, 0)),
            pl.BlockSpec((1, _D), lambda i: (0, 0)),
        ],
        out_specs=pl.BlockSpec((_BLK, _D), lambda i: (i, 0)),
        out_shape=jax.ShapeDtypeStruct((_N, _D), jnp.bfloat16),
    )(x, W_pool.T, b_pool.reshape(1, _D))

    h32 = lax.bitcast_convert_type(h.reshape(_N, _D // 2, 2),
                                   jnp.int32)
    neigh = _sc_seg_max(h32, src, dst, w).reshape(_NPAD, _D)[:_N]

    out = pl.pallas_call(
        _final_body,
        grid=(_N // _BLK,),
        in_specs=[
            pl.BlockSpec((_BLK, _D), lambda i: (i, 0)),
            pl.BlockSpec((_BLK, _D), lambda i: (i, 0)),
            pl.BlockSpec((_D, _D), lambda i: (0, 0)),
            pl.BlockSpec((_D, _D), lambda i: (0, 0)),
            pl.BlockSpec((1, _D), lambda i: (0, 0)),
        ],
        out_specs=pl.BlockSpec((_BLK, _D), lambda i: (i, 0)),
        out_shape=jax.ShapeDtypeStruct((_N, _D), jnp.float32),
    )(x, neigh, W_self.T, W_neigh.T.astype(jnp.bfloat16),
      (b_self + b_neigh).reshape(1, _D))
    return out
